# baseline (speedup 1.0000x reference)
import functools

import jax
import jax.numpy as jnp
from jax import lax
from jax.experimental import pallas as pl
from jax.experimental.pallas import tpu as pltpu
from jax.experimental.pallas import tpu_sc as plsc

B = 32
N = 100
C = 34
HW = 128 * 128
K = N * C
KP = 3456
CHUNK = 128
NCHUNK = KP // CHUNK
WIN = KP + CHUNK
AMAX = B * K - WIN
NFULL = (K // 16)


@functools.partial(
    pl.kernel,
    out_type=jax.ShapeDtypeStruct((B, 128), jnp.float32),
    mesh=plsc.VectorSubcoreMesh(core_axis_name="c", subcore_axis_name="s"),
    compiler_params=pltpu.CompilerParams(needs_layout_passes=False),
    scratch_types=[
        pltpu.VMEM((B * N,), jnp.int32),
        pltpu.VMEM((KP,), jnp.int32),
        pltpu.VMEM((KP,), jnp.float32),
        pltpu.VMEM((WIN,), jnp.float32),
        pltpu.VMEM((WIN,), jnp.int32),
        pltpu.VMEM((128,), jnp.float32),
        pltpu.SemaphoreType.DMA,
        pltpu.SemaphoreType.DMA,
        pltpu.SemaphoreType.DMA,
    ],
)
def _sc_loss(feat_hbm, ind_hbm, tgt_hbm, msk_hbm, out_hbm,
             ind_v, idx_v, pred_v, tgt_v, msk_v, out_v, sem, sem_ind, sem_in):
    b = lax.axis_index("s") * 2 + lax.axis_index("c")
    lane = lax.iota(jnp.int32, 16)
    zf = jnp.zeros((16,), jnp.float32)
    cvec = jnp.full((16,), C, jnp.int32)
    nmax = jnp.full((16,), N - 1, jnp.int32)

    start = b * K
    astart = jnp.minimum(lax.div(start, 128) * 128, AMAX)
    astart = pl.multiple_of(astart, 128)
    phase = start - astart

    pltpu.async_copy(ind_hbm, ind_v, sem_ind)
    pltpu.async_copy(tgt_hbm.at[pl.ds(astart, WIN)], tgt_v, sem_in)
    pltpu.async_copy(msk_hbm.at[pl.ds(astart, WIN)], msk_v, sem_in)
    pltpu.make_async_copy(ind_hbm, ind_v, sem_ind).wait()

    bbase = b * (C * HW)
    nbase = b * N

    def build(k, _):
        p = lane + k * 16
        n = lax.div(p, cvec)
        c = p - n * cvec
        n = jnp.minimum(n, nmax)
        base = plsc.load_gather(ind_v, [nbase + n])
        idx_v[pl.ds(pl.multiple_of(k * 16, 16), 16)] = bbase + c * HW + base
        return 0

    lax.fori_loop(0, KP // 16, build, 0, unroll=4)

    def fire(m, _):
        off = pl.multiple_of(m * CHUNK, CHUNK)
        pltpu.async_copy(
            feat_hbm.at[idx_v.at[pl.ds(off, CHUNK)]],
            pred_v.at[pl.ds(off, CHUNK)],
            sem,
        )
        return 0

    lax.fori_loop(0, NCHUNK, fire, 0)
    pltpu.make_async_copy(feat_hbm.at[pl.ds(0, KP)], pred_v, sem).wait()
    pltpu.make_async_copy(tgt_hbm.at[pl.ds(0, WIN)], tgt_v, sem_in).wait()
    pltpu.make_async_copy(msk_hbm.at[pl.ds(0, WIN)], msk_v, sem_in).wait()

    def body(k, carry):
        accn, accd = carry
        o = pl.multiple_of(k * 16, 16)
        g = phase + o + lane
        pv = pred_v[pl.ds(o, 16)]
        tv = plsc.load_gather(tgt_v, [g])
        mv = plsc.load_gather(msk_v, [g]).astype(jnp.float32)
        accn = accn + jnp.abs(pv - tv) * mv
        accd = accd + mv
        return accn, accd

    accn, accd = lax.fori_loop(0, NFULL, body, (zf, zf), unroll=4)

    valid = lane < 8
    g = jnp.where(valid, phase + NFULL * 16 + lane, phase)
    pv = pred_v[pl.ds(NFULL * 16, 16)]
    tv = plsc.load_gather(tgt_v, [g])
    mv = jnp.where(valid, plsc.load_gather(msk_v, [g]).astype(jnp.float32), 0.0)
    accn = accn + jnp.abs(pv - tv) * mv
    accd = accd + mv

    n_s = jnp.sum(accn)
    d_s = jnp.sum(accd)
    out_v[pl.ds(0, 16)] = jnp.where(lane == 0, n_s, 0.0) + jnp.where(lane == 1, d_s, 0.0)
    for off in range(16, 128, 16):
        out_v[pl.ds(off, 16)] = zf
    pltpu.sync_copy(out_v, out_hbm.at[b])


def kernel(output, mask, ind, target):
    feat = output.reshape(-1)
    ind_f = ind.reshape(-1).astype(jnp.int32)
    tgt_f = target.reshape(-1)
    msk_f = mask.reshape(-1)
    parts = _sc_loss(feat, ind_f, tgt_f, msk_f)
    return jnp.sum(parts[:, 0]) / (jnp.sum(parts[:, 1]) + 0.0001)


# speedup vs baseline: 1.1146x; 1.1146x over previous
import functools

import jax
import jax.numpy as jnp
from jax import lax
from jax.experimental import pallas as pl
from jax.experimental.pallas import tpu as pltpu
from jax.experimental.pallas import tpu_sc as plsc

B = 32
N = 100
C = 34
HW = 128 * 128
K = N * C
KP = 3456
NPAD = 128
CHUNK = 128
NCHUNK = KP // CHUNK
NGROUP = 3
GCHUNK = NCHUNK // NGROUP
GELEM = GCHUNK * CHUNK


@functools.partial(
    pl.kernel,
    out_type=jax.ShapeDtypeStruct((B, 128), jnp.float32),
    mesh=plsc.VectorSubcoreMesh(core_axis_name="c", subcore_axis_name="s"),
    compiler_params=pltpu.CompilerParams(needs_layout_passes=False),
    scratch_types=[
        pltpu.VMEM((NPAD,), jnp.int32),
        pltpu.VMEM((KP,), jnp.int32),
        pltpu.VMEM((KP,), jnp.float32),
        pltpu.VMEM((KP,), jnp.float32),
        pltpu.VMEM((KP,), jnp.float32),
        pltpu.VMEM((128,), jnp.float32),
        pltpu.SemaphoreType.DMA,
        pltpu.SemaphoreType.DMA,
        pltpu.SemaphoreType.DMA,
        pltpu.SemaphoreType.DMA,
        pltpu.SemaphoreType.DMA,
    ],
)
def _sc_loss(feat_hbm, ind_hbm, tgt_hbm, msk_hbm, out_hbm,
             ind_v, idx_v, pred_v, tgt_v, msk_v, out_v,
             sg0, sg1, sg2, sem_ind, sem_in):
    b = lax.axis_index("s") * 2 + lax.axis_index("c")
    lane = lax.iota(jnp.int32, 16)
    zf = jnp.zeros((16,), jnp.float32)
    cvec = jnp.full((16,), C, jnp.int32)
    nmax = jnp.full((16,), N - 1, jnp.int32)
    groups = (sg0, sg1, sg2)

    pltpu.async_copy(ind_hbm.at[b], ind_v, sem_ind)
    pltpu.async_copy(tgt_hbm.at[b], tgt_v, sem_in)
    pltpu.async_copy(msk_hbm.at[b], msk_v, sem_in)
    pltpu.make_async_copy(ind_hbm.at[b], ind_v, sem_ind).wait()

    bbase = b * (C * HW)

    def make_build_fire(sem):
        def build_fire(m, _):
            coff = pl.multiple_of(m * CHUNK, CHUNK)
            for j in range(CHUNK // 16):
                p = lane + (coff + j * 16)
                n = lax.div(p, cvec)
                c = p - n * cvec
                n = jnp.minimum(n, nmax)
                base = plsc.load_gather(ind_v, [n])
                idx_v[pl.ds(coff + j * 16, 16)] = bbase + c * HW + base
            pltpu.async_copy(
                feat_hbm.at[idx_v.at[pl.ds(coff, CHUNK)]],
                pred_v.at[pl.ds(coff, CHUNK)],
                sem,
            )
            return 0
        return build_fire

    for g in range(NGROUP):
        lax.fori_loop(g * GCHUNK, (g + 1) * GCHUNK, make_build_fire(groups[g]), 0)

    pltpu.make_async_copy(tgt_hbm.at[b], tgt_v, sem_in).wait()
    pltpu.make_async_copy(msk_hbm.at[b], msk_v, sem_in).wait()

    def body(k, carry):
        accn, accd = carry
        o = pl.multiple_of(k * 16, 16)
        pv = pred_v[pl.ds(o, 16)]
        tv = tgt_v[pl.ds(o, 16)]
        mv = msk_v[pl.ds(o, 16)]
        accn = accn + jnp.abs(pv - tv) * mv
        accd = accd + mv
        return accn, accd

    accn, accd = zf, zf
    for g in range(NGROUP):
        pltpu.make_async_copy(
            feat_hbm.at[pl.ds(0, GELEM)],
            pred_v.at[pl.ds(g * GELEM, GELEM)],
            groups[g],
        ).wait()
        accn, accd = lax.fori_loop(
            g * (GELEM // 16), (g + 1) * (GELEM // 16), body, (accn, accd),
            unroll=4)

    out_v[pl.ds(0, 16)] = accn
    out_v[pl.ds(16, 16)] = accd
    pltpu.sync_copy(out_v, out_hbm.at[b])


def kernel(output, mask, ind, target):
    feat = output.reshape(-1)
    ind_p = jnp.pad(ind.astype(jnp.int32), ((0, 0), (0, NPAD - N)))
    tgt_p = jnp.pad(target.reshape(B, K), ((0, 0), (0, KP - K)))
    msk_p = jnp.pad(mask.reshape(B, K).astype(jnp.float32), ((0, 0), (0, KP - K)))
    parts = _sc_loss(feat, ind_p, tgt_p, msk_p)
    return jnp.sum(parts[:, 0:16]) / (jnp.sum(parts[:, 16:32]) + 0.0001)


# speedup vs baseline: 1.1182x; 1.0032x over previous
import functools

import jax
import jax.numpy as jnp
from jax import lax
from jax.experimental import pallas as pl
from jax.experimental.pallas import tpu as pltpu
from jax.experimental.pallas import tpu_sc as plsc

B = 32
N = 100
C = 34
HW = 128 * 128
K = N * C
KP = 3456
NPAD = 128
CHUNK = 128
NCHUNK = KP // CHUNK
NGROUP = 3
GCHUNK = NCHUNK // NGROUP
GELEM = GCHUNK * CHUNK


@functools.partial(
    pl.kernel,
    out_type=jax.ShapeDtypeStruct((B, 128), jnp.float32),
    mesh=plsc.VectorSubcoreMesh(core_axis_name="c", subcore_axis_name="s"),
    compiler_params=pltpu.CompilerParams(
        needs_layout_passes=False,
        skip_device_barrier=True,
        disable_bounds_checks=True,
        disable_semaphore_checks=True,
    ),
    scratch_types=[
        pltpu.VMEM((NPAD,), jnp.int32),
        pltpu.VMEM((KP,), jnp.int32),
        pltpu.VMEM((KP,), jnp.float32),
        pltpu.VMEM((KP,), jnp.float32),
        pltpu.VMEM((KP,), jnp.float32),
        pltpu.VMEM((128,), jnp.float32),
        pltpu.SemaphoreType.DMA,
        pltpu.SemaphoreType.DMA,
        pltpu.SemaphoreType.DMA,
        pltpu.SemaphoreType.DMA,
        pltpu.SemaphoreType.DMA,
    ],
)
def _sc_loss(feat_hbm, ind_hbm, tgt_hbm, msk_hbm, out_hbm,
             ind_v, idx_v, pred_v, tgt_v, msk_v, out_v,
             sg0, sg1, sg2, sem_ind, sem_in):
    b = lax.axis_index("s") * 2 + lax.axis_index("c")
    lane = lax.iota(jnp.int32, 16)
    zf = jnp.zeros((16,), jnp.float32)
    cvec = jnp.full((16,), C, jnp.int32)
    nmax = jnp.full((16,), N - 1, jnp.int32)
    groups = (sg0, sg1, sg2)

    pltpu.async_copy(ind_hbm.at[b], ind_v, sem_ind)
    pltpu.async_copy(tgt_hbm.at[b], tgt_v, sem_in)
    pltpu.async_copy(msk_hbm.at[b], msk_v, sem_in)
    pltpu.make_async_copy(ind_hbm.at[b], ind_v, sem_ind).wait()

    bbase = b * (C * HW)

    def make_build_fire(sem):
        def build_fire(m, _):
            coff = pl.multiple_of(m * CHUNK, CHUNK)
            for j in range(CHUNK // 16):
                p = lane + (coff + j * 16)
                n = lax.div(p, cvec)
                c = p - n * cvec
                n = jnp.minimum(n, nmax)
                base = plsc.load_gather(ind_v, [n])
                idx_v[pl.ds(coff + j * 16, 16)] = bbase + c * HW + base
            pltpu.async_copy(
                feat_hbm.at[idx_v.at[pl.ds(coff, CHUNK)]],
                pred_v.at[pl.ds(coff, CHUNK)],
                sem,
            )
            return 0
        return build_fire

    for g in range(NGROUP):
        lax.fori_loop(g * GCHUNK, (g + 1) * GCHUNK, make_build_fire(groups[g]), 0)

    pltpu.make_async_copy(tgt_hbm.at[b], tgt_v, sem_in).wait()
    pltpu.make_async_copy(msk_hbm.at[b], msk_v, sem_in).wait()

    def body(k, carry):
        accn, accd = carry
        o = pl.multiple_of(k * 16, 16)
        pv = pred_v[pl.ds(o, 16)]
        tv = tgt_v[pl.ds(o, 16)]
        mv = msk_v[pl.ds(o, 16)]
        accn = accn + jnp.abs(pv - tv) * mv
        accd = accd + mv
        return accn, accd

    accn, accd = zf, zf
    for g in range(NGROUP):
        pltpu.make_async_copy(
            feat_hbm.at[pl.ds(0, GELEM)],
            pred_v.at[pl.ds(g * GELEM, GELEM)],
            groups[g],
        ).wait()
        accn, accd = lax.fori_loop(
            g * (GELEM // 16), (g + 1) * (GELEM // 16), body, (accn, accd),
            unroll=4)

    out_v[pl.ds(0, 16)] = accn
    out_v[pl.ds(16, 16)] = accd
    pltpu.sync_copy(out_v, out_hbm.at[b])


def kernel(output, mask, ind, target):
    feat = output.reshape(-1)
    ind_p = jnp.pad(ind.astype(jnp.int32), ((0, 0), (0, NPAD - N)))
    tgt_p = jnp.pad(target.reshape(B, K), ((0, 0), (0, KP - K)))
    msk_p = jnp.pad(mask.reshape(B, K).astype(jnp.float32), ((0, 0), (0, KP - K)))
    parts = _sc_loss(feat, ind_p, tgt_p, msk_p)
    return jnp.sum(parts[:, 0:16]) / (jnp.sum(parts[:, 16:32]) + 0.0001)
